# baseline (device time: 104306 ns/iter reference)
import jax
import jax.numpy as jnp
from jax import lax
from jax.experimental import pallas as pl
from jax.experimental.pallas import tpu as pltpu

N_DEV = 16
CLIP = 5.0


def kernel(A, B):
    m_per, k = A.shape
    _, n = B.shape

    def body(a_ref, b_ref, out_ref, gathered, send_sems, recv_sems):
        my = lax.axis_index("i")
        q = lax.rem(my, 4)
        z = lax.div(my, 4)
        b = lax.rem(z, 2) * 2 + lax.div(z, 2)
        my_slot = q * 4 + b

        def origin_of(local_slot):
            g = local_slot ^ my_slot
            qq = lax.div(g, 4)
            bb = lax.rem(g, 4)
            zz = lax.rem(bb, 2) * 2 + lax.div(bb, 2)
            return 4 * zz + qq

        pA = 4 * (z ^ 2) + q
        pB = 4 * (z ^ 1) + q
        pC = 4 * z + (q ^ 1)
        pD = 4 * z + (q ^ 3)

        a_q = jnp.clip(
            jnp.round(a_ref[...] * (127.0 / CLIP)), -127.0, 127.0
        ).astype(jnp.int8)
        gathered[0] = a_q
        b_scaled = (b_ref[...] * (CLIP / 127.0)).astype(jnp.bfloat16)

        barrier_sem = pltpu.get_barrier_semaphore()
        for p in (pA, pB, pC, pD):
            pl.semaphore_signal(
                barrier_sem, inc=1,
                device_id=(p,), device_id_type=pl.DeviceIdType.MESH,
            )
        pl.semaphore_wait(barrier_sem, 4)

        def dot_store(origin, chunk):
            out_ref[pl.ds(origin * m_per, m_per), :] = jnp.dot(
                chunk.astype(jnp.bfloat16), b_scaled,
                preferred_element_type=jnp.float32,
            )

        def exchange(step, src_lo, size, dst_lo, partner):
            snd = pltpu.make_async_remote_copy(
                src_ref=gathered.at[pl.ds(src_lo, size)],
                dst_ref=gathered.at[pl.ds(dst_lo, size)],
                send_sem=send_sems.at[step],
                recv_sem=recv_sems.at[step],
                device_id=(partner,), device_id_type=pl.DeviceIdType.MESH,
            )
            snd.start()
            rcv = pltpu.make_async_remote_copy(
                src_ref=gathered.at[pl.ds(src_lo, size)],
                dst_ref=gathered.at[pl.ds(dst_lo, size)],
                send_sem=send_sems.at[step],
                recv_sem=recv_sems.at[step],
                device_id=(partner,), device_id_type=pl.DeviceIdType.MESH,
            )
            return snd, rcv

        sA, rA = exchange(0, 0, 1, 1, pA)
        dot_store(my, a_q)
        rA.wait_recv()

        sB, rB = exchange(1, 0, 2, 2, pB)
        dot_store(origin_of(1), gathered[1])
        rB.wait_recv()

        sC, rC = exchange(2, 0, 4, 4, pC)
        for l in (2, 3):
            dot_store(origin_of(l), gathered[l])
        rC.wait_recv()

        sD1, rD1 = exchange(3, 0, 4, 12, pD)
        sD2, rD2 = exchange(4, 4, 4, 8, pD)
        for l in (4, 5, 6, 7):
            dot_store(origin_of(l), gathered[l])
        rD1.wait_recv()
        for l in (12, 13, 14, 15):
            dot_store(origin_of(l), gathered[l])
        rD2.wait_recv()
        for l in (8, 9, 10, 11):
            dot_store(origin_of(l), gathered[l])

        for snd in (sA, sB, sC, sD1, sD2):
            snd.wait_send()

    return pl.pallas_call(
        body,
        out_shape=jax.ShapeDtypeStruct((N_DEV * m_per, n), jnp.float32),
        in_specs=[
            pl.BlockSpec(memory_space=pltpu.VMEM),
            pl.BlockSpec(memory_space=pltpu.VMEM),
        ],
        out_specs=pl.BlockSpec(memory_space=pltpu.VMEM),
        scratch_shapes=[
            pltpu.VMEM((N_DEV, m_per, k), jnp.int8),
            pltpu.SemaphoreType.DMA((5,)),
            pltpu.SemaphoreType.DMA((5,)),
        ],
        compiler_params=pltpu.CompilerParams(
            collective_id=0, vmem_limit_bytes=100 * 1024 * 1024
        ),
    )(A, B)


# device time: 83769 ns/iter; 1.2452x vs baseline; 1.2452x over previous
import jax
import jax.numpy as jnp
from jax import lax
from jax.experimental import pallas as pl
from jax.experimental.pallas import tpu as pltpu

N_DEV = 16
HR = N_DEV // 2
HL = N_DEV // 2 - 1
NSLOT = 4
CLIP = 5.0


def kernel(A, B):
    m_per, k = A.shape
    _, n = B.shape

    def body(a_ref, b_ref, out_ref,
             commR, commL, sendR, recvR, sendL, recvL, ackR, ackL):
        my = lax.axis_index("i")
        q = lax.rem(my, 4)
        z = lax.div(my, 4)
        ci = jnp.where(
            q == 1, 1 + z,
            jnp.where(q == 2, 8 - z,
                      jnp.where(q == 3, 9 + z,
                                jnp.where(z == 0, 0, 16 - z))),
        ).astype(jnp.int32)

        def cyc(j):
            return jnp.where(
                j == 0, 0,
                jnp.where(j <= 4, 4 * j - 3,
                          jnp.where(j <= 8, 34 - 4 * j,
                                    jnp.where(j <= 12, 4 * j - 33,
                                              64 - 4 * j))),
            ).astype(jnp.int32)

        left = cyc(lax.rem(ci + N_DEV - 1, N_DEV))
        right = cyc(lax.rem(ci + 1, N_DEV))

        a_q = jnp.clip(
            jnp.round(a_ref[...] * (127.0 / CLIP)), -127.0, 127.0
        ).astype(jnp.int8)
        commR[NSLOT - 1] = a_q
        commL[NSLOT - 1] = a_q
        b_scaled = (b_ref[...] * (CLIP / 127.0)).astype(jnp.bfloat16)

        barrier_sem = pltpu.get_barrier_semaphore()
        for nbr in (left, right):
            pl.semaphore_signal(
                barrier_sem, inc=1,
                device_id=(nbr,), device_id_type=pl.DeviceIdType.MESH,
            )
        pl.semaphore_wait(barrier_sem, 2)

        def dot_store(origin, chunk):
            out_ref[pl.ds(origin * m_per, m_per), :] = jnp.dot(
                chunk.astype(jnp.bfloat16), b_scaled,
                preferred_element_type=jnp.float32,
            )

        def send(comm, send_sems, recv_sems, h, nbr):
            s = (h + NSLOT - 1) % NSLOT
            r = h % NSLOT
            rdma = pltpu.make_async_remote_copy(
                src_ref=comm.at[s], dst_ref=comm.at[r],
                send_sem=send_sems.at[s], recv_sem=recv_sems.at[r],
                device_id=(nbr,), device_id_type=pl.DeviceIdType.MESH,
            )
            rdma.start()
            return rdma

        rdmaR = send(commR, sendR, recvR, 0, right)
        rdmaL = send(commL, sendL, recvL, 0, left)
        dot_store(my, a_q)
        rdmaR.wait_send()
        pl.semaphore_signal(ackR, inc=1, device_id=(left,),
                            device_id_type=pl.DeviceIdType.MESH)
        rdmaL.wait_send()
        pl.semaphore_signal(ackL, inc=1, device_id=(right,),
                            device_id_type=pl.DeviceIdType.MESH)

        for h in range(1, HR):
            rdmaR.wait_recv()
            if h >= NSLOT - 1:
                pl.semaphore_wait(ackR, 1)
            rdmaR = send(commR, sendR, recvR, h, right)
            if h - 1 < HL:
                rdmaL.wait_recv()
                if h < HL:
                    if h >= NSLOT - 1:
                        pl.semaphore_wait(ackL, 1)
                    rdmaL = send(commL, sendL, recvL, h, left)

            s = (h + NSLOT - 1) % NSLOT
            dot_store(cyc(lax.rem(ci + N_DEV - h, N_DEV)), commR[s])
            dot_store(cyc(lax.rem(ci + h, N_DEV)), commL[s])

            rdmaR.wait_send()
            if h <= HR - NSLOT:
                pl.semaphore_signal(ackR, inc=1, device_id=(left,),
                                    device_id_type=pl.DeviceIdType.MESH)
            if h < HL:
                rdmaL.wait_send()
                if h <= HL - NSLOT:
                    pl.semaphore_signal(ackL, inc=1, device_id=(right,),
                                        device_id_type=pl.DeviceIdType.MESH)

        rdmaR.wait_recv()
        dot_store(cyc(lax.rem(ci + N_DEV - HR, N_DEV)), commR[(HR - 1) % NSLOT])

    return pl.pallas_call(
        body,
        out_shape=jax.ShapeDtypeStruct((N_DEV * m_per, n), jnp.float32),
        in_specs=[
            pl.BlockSpec(memory_space=pltpu.VMEM),
            pl.BlockSpec(memory_space=pltpu.VMEM),
        ],
        out_specs=pl.BlockSpec(memory_space=pltpu.VMEM),
        scratch_shapes=[
            pltpu.VMEM((NSLOT, m_per, k), jnp.int8),
            pltpu.VMEM((NSLOT, m_per, k), jnp.int8),
            pltpu.SemaphoreType.DMA((NSLOT,)),
            pltpu.SemaphoreType.DMA((NSLOT,)),
            pltpu.SemaphoreType.DMA((NSLOT,)),
            pltpu.SemaphoreType.DMA((NSLOT,)),
            pltpu.SemaphoreType.REGULAR,
            pltpu.SemaphoreType.REGULAR,
        ],
        compiler_params=pltpu.CompilerParams(
            collective_id=0, vmem_limit_bytes=100 * 1024 * 1024
        ),
    )(A, B)


# device time: 79479 ns/iter; 1.3124x vs baseline; 1.0540x over previous
import jax
import jax.numpy as jnp
from jax import lax
from jax.experimental import pallas as pl
from jax.experimental.pallas import tpu as pltpu

N_DEV = 16
HR = N_DEV // 2
HL = N_DEV // 2 - 1
NSLOT = 4
CLIP = 5.0


def kernel(A, B):
    m_per, k = A.shape
    _, n = B.shape

    def body(a_ref, b_ref, out_ref,
             commR, commL, sendR, recvR, sendL, recvL, ackR, ackL):
        my = lax.axis_index("i")
        q = lax.rem(my, 4)
        z = lax.div(my, 4)
        ci = jnp.where(
            q == 1, 1 + z,
            jnp.where(q == 2, 8 - z,
                      jnp.where(q == 3, 9 + z,
                                jnp.where(z == 0, 0, 16 - z))),
        ).astype(jnp.int32)

        def cyc(j):
            return jnp.where(
                j == 0, 0,
                jnp.where(j <= 4, 4 * j - 3,
                          jnp.where(j <= 8, 34 - 4 * j,
                                    jnp.where(j <= 12, 4 * j - 33,
                                              64 - 4 * j))),
            ).astype(jnp.int32)

        left = cyc(lax.rem(ci + N_DEV - 1, N_DEV))
        right = cyc(lax.rem(ci + 1, N_DEV))

        a_q = jnp.clip(
            jnp.round(a_ref[...] * (127.0 / CLIP)), -127.0, 127.0
        ).astype(jnp.int8)
        commR[NSLOT - 1] = a_q
        commL[NSLOT - 1] = a_q
        b_scaled = (b_ref[...] * (CLIP / 127.0)).astype(jnp.bfloat16)

        barrier_sem = pltpu.get_barrier_semaphore()
        for nbr in (left, right):
            pl.semaphore_signal(
                barrier_sem, inc=1,
                device_id=(nbr,), device_id_type=pl.DeviceIdType.MESH,
            )
        pl.semaphore_wait(barrier_sem, 2)

        def dot_store(origin, chunk):
            out_ref[pl.ds(origin * m_per, m_per), :] = jnp.dot(
                chunk.astype(jnp.bfloat16), b_scaled,
                preferred_element_type=jnp.float32,
            )

        hm = m_per // 2

        def send_half(comm, send_sems, recv_sems, h, nbr, half):
            s = (h + NSLOT - 1) % NSLOT
            r = h % NSLOT
            rdma = pltpu.make_async_remote_copy(
                src_ref=comm.at[s, pl.ds(half * hm, hm)],
                dst_ref=comm.at[r, pl.ds(half * hm, hm)],
                send_sem=send_sems.at[2 * s + half],
                recv_sem=recv_sems.at[2 * r + half],
                device_id=(nbr,), device_id_type=pl.DeviceIdType.MESH,
            )
            rdma.start()
            return rdma

        rR = [send_half(commR, sendR, recvR, 0, right, hf) for hf in (0, 1)]
        rL = [send_half(commL, sendL, recvL, 0, left, hf) for hf in (0, 1)]
        dot_store(my, a_q)
        for rdma in rR:
            rdma.wait_send()
        pl.semaphore_signal(ackR, inc=1, device_id=(left,),
                            device_id_type=pl.DeviceIdType.MESH)
        for rdma in rL:
            rdma.wait_send()
        pl.semaphore_signal(ackL, inc=1, device_id=(right,),
                            device_id_type=pl.DeviceIdType.MESH)

        for h in range(1, HR):
            rR[0].wait_recv()
            if h >= NSLOT - 1:
                pl.semaphore_wait(ackR, 1)
            nR0 = send_half(commR, sendR, recvR, h, right, 0)
            rR[1].wait_recv()
            nR1 = send_half(commR, sendR, recvR, h, right, 1)
            rR = [nR0, nR1]
            if h - 1 < HL:
                rL[0].wait_recv()
                rL1_prev = rL[1]
                if h < HL:
                    if h >= NSLOT - 1:
                        pl.semaphore_wait(ackL, 1)
                    nL0 = send_half(commL, sendL, recvL, h, left, 0)
                    rL1_prev.wait_recv()
                    nL1 = send_half(commL, sendL, recvL, h, left, 1)
                    rL = [nL0, nL1]
                else:
                    rL1_prev.wait_recv()

            s = (h + NSLOT - 1) % NSLOT
            dot_store(cyc(lax.rem(ci + N_DEV - h, N_DEV)), commR[s])
            dot_store(cyc(lax.rem(ci + h, N_DEV)), commL[s])

            rR[0].wait_send()
            rR[1].wait_send()
            if h <= HR - NSLOT:
                pl.semaphore_signal(ackR, inc=1, device_id=(left,),
                                    device_id_type=pl.DeviceIdType.MESH)
            if h < HL:
                rL[0].wait_send()
                rL[1].wait_send()
                if h <= HL - NSLOT:
                    pl.semaphore_signal(ackL, inc=1, device_id=(right,),
                                        device_id_type=pl.DeviceIdType.MESH)

        rR[0].wait_recv()
        rR[1].wait_recv()
        dot_store(cyc(lax.rem(ci + N_DEV - HR, N_DEV)), commR[(HR - 1) % NSLOT])

    return pl.pallas_call(
        body,
        out_shape=jax.ShapeDtypeStruct((N_DEV * m_per, n), jnp.float32),
        in_specs=[
            pl.BlockSpec(memory_space=pltpu.VMEM),
            pl.BlockSpec(memory_space=pltpu.VMEM),
        ],
        out_specs=pl.BlockSpec(memory_space=pltpu.VMEM),
        scratch_shapes=[
            pltpu.VMEM((NSLOT, m_per, k), jnp.int8),
            pltpu.VMEM((NSLOT, m_per, k), jnp.int8),
            pltpu.SemaphoreType.DMA((2 * NSLOT,)),
            pltpu.SemaphoreType.DMA((2 * NSLOT,)),
            pltpu.SemaphoreType.DMA((2 * NSLOT,)),
            pltpu.SemaphoreType.DMA((2 * NSLOT,)),
            pltpu.SemaphoreType.REGULAR,
            pltpu.SemaphoreType.REGULAR,
        ],
        compiler_params=pltpu.CompilerParams(
            collective_id=0, vmem_limit_bytes=100 * 1024 * 1024
        ),
    )(A, B)
